# baseline (speedup 1.0000x reference)
import jax
import jax.numpy as jnp
from jax.experimental import pallas as pl
from jax.experimental.pallas import tpu as pltpu

_N = 4096
_B = 4
_K = 5
_F1 = 20
_F2 = 30
_C = 10
_BR = 512

_HP = jax.lax.Precision.HIGHEST


def _lmul_first(l_ref, x_ref, o_ref):
    o_ref[...] = jax.lax.dot(l_ref[...], x_ref[...], precision=_HP)


def _lmul_step(l_ref, x_ref, p_ref, o_ref):
    o_ref[...] = (2.0 * jax.lax.dot(l_ref[...], x_ref[...], precision=_HP)
                  - p_ref[...])


def _lmul(L, x, prev=None):
    W = x.shape[1]
    grid = (_N // _BR,)
    l_spec = pl.BlockSpec((_BR, _N), lambda i: (i, 0))
    x_spec = pl.BlockSpec((_N, W), lambda i: (0, 0))
    o_spec = pl.BlockSpec((_BR, W), lambda i: (i, 0))
    out_shape = jax.ShapeDtypeStruct((_N, W), jnp.float32)
    if prev is None:
        return pl.pallas_call(
            _lmul_first, grid=grid, in_specs=[l_spec, x_spec],
            out_specs=o_spec, out_shape=out_shape,
        )(L, x)
    return pl.pallas_call(
        _lmul_step, grid=grid, in_specs=[l_spec, x_spec, o_spec],
        out_specs=o_spec, out_shape=out_shape,
    )(L, x, prev)


def _combine_kernel(t_ref, m_ref, b_ref, o_ref):
    acc = jax.lax.dot(t_ref[...], m_ref[...], precision=_HP)
    o_ref[...] = jnp.maximum(acc + b_ref[...], 0.0)


def _combine(Tcat, M, bt):
    KW = Tcat.shape[1]
    G = M.shape[1]
    return pl.pallas_call(
        _combine_kernel,
        out_shape=jax.ShapeDtypeStruct((_N, G), jnp.float32),
    )(Tcat, M, bt)


def _fc_kernel(h_ref, wfc_ref, msk_ref, sb_ref, sc_ref, bfc_ref, o_ref):
    U = jax.lax.dot_general(h_ref[...], wfc_ref[...],
                            dimension_numbers=(((0,), (0,)), ((), ())),
                            precision=_HP)
    Um = U * msk_ref[...]
    logits = jax.lax.dot(sb_ref[...],
                         jax.lax.dot(Um, sc_ref[...], precision=_HP),
                         precision=_HP) + bfc_ref[...]
    m = jnp.max(logits, axis=1, keepdims=True)
    z = logits - m
    lse = jnp.log(jnp.sum(jnp.exp(z), axis=1, keepdims=True))
    o_ref[...] = z - lse


def kernel(x, L, W1, b1, W2, b2, Wfc, bfc):
    B, N = _B, _N
    X0 = x[:, :, 0].T

    eyeB = jnp.eye(B, dtype=jnp.float32)
    M1 = (W1[:, 0, :][:, None, None, :] * eyeB[None, :, :, None]
          ).reshape(_K * B, B * _F1)
    M2 = (W2[:, None, :, None, :] * eyeB[None, :, None, :, None]
          ).reshape(_K * B * _F1, B * _F2)
    b1t = jnp.tile(b1, (B,))[None, :]
    b2t = jnp.tile(b2, (B,))[None, :]

    Wfcf = Wfc.reshape(_C, N, _F2).transpose(1, 0, 2).reshape(N, _C * _F2)

    r = jnp.arange(B * _F2)[:, None]
    q = jnp.arange(_C * _F2)[None, :]
    msk = ((r % _F2) == (q % _F2)).astype(jnp.float32)
    sb = (jnp.arange(B)[:, None] == (jnp.arange(B * _F2)[None, :] // _F2)
          ).astype(jnp.float32)
    sc = ((jnp.arange(_C * _F2)[:, None] // _F2) == jnp.arange(_C)[None, :]
          ).astype(jnp.float32)
    bfcr = bfc[None, :]

    T1 = _lmul(L, X0)
    T2 = _lmul(L, T1, X0)
    T3 = _lmul(L, T2, T1)
    T4 = _lmul(L, T3, T2)
    Tcat = jnp.concatenate([X0, T1, T2, T3, T4], axis=1)
    H = _combine(Tcat, M1, b1t)

    S1 = _lmul(L, H)
    S2 = _lmul(L, S1, H)
    S3 = _lmul(L, S2, S1)
    S4 = _lmul(L, S3, S2)
    Scat = jnp.concatenate([H, S1, S2, S3, S4], axis=1)
    h2 = _combine(Scat, M2, b2t)

    out = pl.pallas_call(
        _fc_kernel,
        out_shape=jax.ShapeDtypeStruct((B, _C), jnp.float32),
    )(h2, Wfcf, msk, sb, sc, bfcr)
    return out


# speedup vs baseline: 2.0030x; 2.0030x over previous
import jax
import jax.numpy as jnp
from jax.experimental import pallas as pl
from jax.experimental.pallas import tpu as pltpu

_N = 4096
_B = 4
_K = 5
_F1 = 20
_F2 = 30
_C = 10
_BR = 512

_HP = jax.lax.Precision.DEFAULT


def _lmul_first(l_ref, x_ref, o_ref):
    o_ref[...] = jax.lax.dot(l_ref[...], x_ref[...], precision=_HP)


def _lmul_step(l_ref, x_ref, p_ref, o_ref):
    o_ref[...] = (2.0 * jax.lax.dot(l_ref[...], x_ref[...], precision=_HP)
                  - p_ref[...])


def _lmul(L, x, prev=None):
    W = x.shape[1]
    grid = (_N // _BR,)
    l_spec = pl.BlockSpec((_BR, _N), lambda i: (i, 0))
    x_spec = pl.BlockSpec((_N, W), lambda i: (0, 0))
    o_spec = pl.BlockSpec((_BR, W), lambda i: (i, 0))
    out_shape = jax.ShapeDtypeStruct((_N, W), jnp.float32)
    if prev is None:
        return pl.pallas_call(
            _lmul_first, grid=grid, in_specs=[l_spec, x_spec],
            out_specs=o_spec, out_shape=out_shape,
        )(L, x)
    return pl.pallas_call(
        _lmul_step, grid=grid, in_specs=[l_spec, x_spec, o_spec],
        out_specs=o_spec, out_shape=out_shape,
    )(L, x, prev)


def _combine_kernel(t_ref, m_ref, b_ref, o_ref):
    acc = jax.lax.dot(t_ref[...], m_ref[...], precision=_HP)
    o_ref[...] = jnp.maximum(acc + b_ref[...], 0.0)


def _combine(Tcat, M, bt):
    KW = Tcat.shape[1]
    G = M.shape[1]
    return pl.pallas_call(
        _combine_kernel,
        out_shape=jax.ShapeDtypeStruct((_N, G), jnp.float32),
    )(Tcat, M, bt)


def _fc_kernel(h_ref, wfc_ref, msk_ref, sb_ref, sc_ref, bfc_ref, o_ref):
    U = jax.lax.dot_general(h_ref[...], wfc_ref[...],
                            dimension_numbers=(((0,), (0,)), ((), ())),
                            precision=_HP)
    Um = U * msk_ref[...]
    logits = jax.lax.dot(sb_ref[...],
                         jax.lax.dot(Um, sc_ref[...], precision=_HP),
                         precision=_HP) + bfc_ref[...]
    m = jnp.max(logits, axis=1, keepdims=True)
    z = logits - m
    lse = jnp.log(jnp.sum(jnp.exp(z), axis=1, keepdims=True))
    o_ref[...] = z - lse


def kernel(x, L, W1, b1, W2, b2, Wfc, bfc):
    B, N = _B, _N
    X0 = x[:, :, 0].T

    eyeB = jnp.eye(B, dtype=jnp.float32)
    M1 = (W1[:, 0, :][:, None, None, :] * eyeB[None, :, :, None]
          ).reshape(_K * B, B * _F1)
    M2 = (W2[:, None, :, None, :] * eyeB[None, :, None, :, None]
          ).reshape(_K * B * _F1, B * _F2)
    b1t = jnp.tile(b1, (B,))[None, :]
    b2t = jnp.tile(b2, (B,))[None, :]

    Wfcf = Wfc.reshape(_C, N, _F2).transpose(1, 0, 2).reshape(N, _C * _F2)

    r = jnp.arange(B * _F2)[:, None]
    q = jnp.arange(_C * _F2)[None, :]
    msk = ((r % _F2) == (q % _F2)).astype(jnp.float32)
    sb = (jnp.arange(B)[:, None] == (jnp.arange(B * _F2)[None, :] // _F2)
          ).astype(jnp.float32)
    sc = ((jnp.arange(_C * _F2)[:, None] // _F2) == jnp.arange(_C)[None, :]
          ).astype(jnp.float32)
    bfcr = bfc[None, :]

    T1 = _lmul(L, X0)
    T2 = _lmul(L, T1, X0)
    T3 = _lmul(L, T2, T1)
    T4 = _lmul(L, T3, T2)
    Tcat = jnp.concatenate([X0, T1, T2, T3, T4], axis=1)
    H = _combine(Tcat, M1, b1t)

    S1 = _lmul(L, H)
    S2 = _lmul(L, S1, H)
    S3 = _lmul(L, S2, S1)
    S4 = _lmul(L, S3, S2)
    Scat = jnp.concatenate([H, S1, S2, S3, S4], axis=1)
    h2 = _combine(Scat, M2, b2t)

    out = pl.pallas_call(
        _fc_kernel,
        out_shape=jax.ShapeDtypeStruct((B, _C), jnp.float32),
    )(h2, Wfcf, msk, sb, sc, bfcr)
    return out


# speedup vs baseline: 2.4340x; 1.2152x over previous
import jax
import jax.numpy as jnp
from jax.experimental import pallas as pl
from jax.experimental.pallas import tpu as pltpu

_N = 4096
_NL = 2048
_NR = _N - _NL
_B = 4
_K = 5
_F1 = 20
_F2 = 30
_C = 10
_BR = 256
_NI = _N // _BR

_HP = jax.lax.Precision.DEFAULT


def _mega_kernel(lres_ref, lstr_ref, x0_ref, m1_ref, m2_ref, b1_ref, b2_ref,
                 wfc_ref, msk_ref, sb_ref, sc_ref, bfc_ref, out_ref,
                 tbuf, hb, sa, sb_s, out2, uacc):
    p = pl.program_id(0)
    i = pl.program_id(1)
    r0 = i * _BR

    def lmul(top, bot):
        ltop = lres_ref[pl.ds(r0, _BR), :]
        return (jax.lax.dot(ltop, top, precision=_HP)
                + jax.lax.dot(lstr_ref[...], bot, precision=_HP))

    m1 = m1_ref[...]
    m2 = m2_ref[...]

    @pl.when(p == 0)
    def _():
        t1 = lmul(x0_ref[0:_NL, :], x0_ref[_NL:_N, :])
        tbuf[pl.ds(r0, _BR), 0:4] = t1
        x0b = x0_ref[pl.ds(r0, _BR), :]
        hb[pl.ds(r0, _BR), :] = (jax.lax.dot(x0b, m1[0:4], precision=_HP)
                                 + jax.lax.dot(t1, m1[4:8], precision=_HP))

    @pl.when(p == 1)
    def _():
        t2 = (2.0 * lmul(tbuf[0:_NL, 0:4], tbuf[_NL:_N, 0:4])
              - x0_ref[pl.ds(r0, _BR), :])
        tbuf[pl.ds(r0, _BR), 4:8] = t2
        hb[pl.ds(r0, _BR), :] += jax.lax.dot(t2, m1[8:12], precision=_HP)

    @pl.when(p == 2)
    def _():
        t3 = (2.0 * lmul(tbuf[0:_NL, 4:8], tbuf[_NL:_N, 4:8])
              - tbuf[pl.ds(r0, _BR), 0:4])
        tbuf[pl.ds(r0, _BR), 8:12] = t3
        hb[pl.ds(r0, _BR), :] += jax.lax.dot(t3, m1[12:16], precision=_HP)

    @pl.when(p == 3)
    def _():
        t4 = (2.0 * lmul(tbuf[0:_NL, 8:12], tbuf[_NL:_N, 8:12])
              - tbuf[pl.ds(r0, _BR), 4:8])
        acc = hb[pl.ds(r0, _BR), :] + jax.lax.dot(t4, m1[16:20],
                                                  precision=_HP)
        hb[pl.ds(r0, _BR), :] = jnp.maximum(acc + b1_ref[...], 0.0)

    @pl.when(p == 4)
    def _():
        s1 = lmul(hb[0:_NL, :], hb[_NL:_N, :])
        sa[pl.ds(r0, _BR), :] = s1
        hblk = hb[pl.ds(r0, _BR), :]
        out2[pl.ds(r0, _BR), :] = (
            jax.lax.dot(hblk, m2[0:80], precision=_HP)
            + jax.lax.dot(s1, m2[80:160], precision=_HP))

    @pl.when(p == 5)
    def _():
        s2 = 2.0 * lmul(sa[0:_NL, :], sa[_NL:_N, :]) - hb[pl.ds(r0, _BR), :]
        sb_s[pl.ds(r0, _BR), :] = s2
        out2[pl.ds(r0, _BR), :] += jax.lax.dot(s2, m2[160:240], precision=_HP)

    @pl.when(p == 6)
    def _():
        s3 = (2.0 * lmul(sb_s[0:_NL, :], sb_s[_NL:_N, :])
              - sa[pl.ds(r0, _BR), :])
        sa[pl.ds(r0, _BR), :] = s3
        out2[pl.ds(r0, _BR), :] += jax.lax.dot(s3, m2[240:320], precision=_HP)

    @pl.when(p == 7)
    def _():
        s4 = (2.0 * lmul(sa[0:_NL, :], sa[_NL:_N, :])
              - sb_s[pl.ds(r0, _BR), :])
        acc = out2[pl.ds(r0, _BR), :] + jax.lax.dot(s4, m2[320:400],
                                                    precision=_HP)
        h2 = jnp.maximum(acc + b2_ref[...], 0.0)
        upart = jax.lax.dot_general(h2, wfc_ref[...],
                                    dimension_numbers=(((0,), (0,)), ((), ())),
                                    precision=_HP)

        @pl.when(i == 0)
        def _():
            uacc[...] = upart

        @pl.when(i > 0)
        def _():
            uacc[...] += upart

        @pl.when(i == _NI - 1)
        def _():
            um = uacc[...] * msk_ref[...]
            logits = jax.lax.dot(
                sb_ref[...], jax.lax.dot(um, sc_ref[...], precision=_HP),
                precision=_HP) + bfc_ref[...]
            m = jnp.max(logits, axis=1, keepdims=True)
            z = logits - m
            lse = jnp.log(jnp.sum(jnp.exp(z), axis=1, keepdims=True))
            out_ref[...] = z - lse


def kernel(x, L, W1, b1, W2, b2, Wfc, bfc):
    B, N = _B, _N
    X0 = x[:, :, 0].T

    eyeB = jnp.eye(B, dtype=jnp.float32)
    M1 = (W1[:, 0, :][:, None, None, :] * eyeB[None, :, :, None]
          ).reshape(_K * B, B * _F1)
    M2 = (W2[:, None, :, None, :] * eyeB[None, :, None, :, None]
          ).reshape(_K * B * _F1, B * _F2)
    b1t = jnp.tile(b1, (B,))[None, :]
    b2t = jnp.tile(b2, (B,))[None, :]

    Wfcf = Wfc.reshape(_C, N, _F2).transpose(1, 0, 2).reshape(N, _C * _F2)

    r = jnp.arange(B * _F2)[:, None]
    q = jnp.arange(_C * _F2)[None, :]
    msk = ((r % _F2) == (q % _F2)).astype(jnp.float32)
    sb = (jnp.arange(B)[:, None] == (jnp.arange(B * _F2)[None, :] // _F2)
          ).astype(jnp.float32)
    sc = ((jnp.arange(_C * _F2)[:, None] // _F2) == jnp.arange(_C)[None, :]
          ).astype(jnp.float32)
    bfcr = bfc[None, :]

    grid = (2 * _K - 2, _NI)
    out = pl.pallas_call(
        _mega_kernel,
        grid=grid,
        in_specs=[
            pl.BlockSpec((_N, _NL), lambda p, i: (0, 0)),
            pl.BlockSpec((_BR, _NR), lambda p, i: (i, 1)),
            pl.BlockSpec((_N, _B), lambda p, i: (0, 0)),
            pl.BlockSpec((_K * _B, _B * _F1), lambda p, i: (0, 0)),
            pl.BlockSpec((_K * _B * _F1, _B * _F2), lambda p, i: (0, 0)),
            pl.BlockSpec((1, _B * _F1), lambda p, i: (0, 0)),
            pl.BlockSpec((1, _B * _F2), lambda p, i: (0, 0)),
            pl.BlockSpec((_BR, _C * _F2),
                         lambda p, i: (jnp.where(p == 7, i, 0), 0)),
            pl.BlockSpec((_B * _F2, _C * _F2), lambda p, i: (0, 0)),
            pl.BlockSpec((_B, _B * _F2), lambda p, i: (0, 0)),
            pl.BlockSpec((_C * _F2, _C), lambda p, i: (0, 0)),
            pl.BlockSpec((1, _C), lambda p, i: (0, 0)),
        ],
        out_specs=pl.BlockSpec((_B, _C), lambda p, i: (0, 0)),
        out_shape=jax.ShapeDtypeStruct((B, _C), jnp.float32),
        scratch_shapes=[
            pltpu.VMEM((_N, 16), jnp.float32),
            pltpu.VMEM((_N, _B * _F1), jnp.float32),
            pltpu.VMEM((_N, _B * _F1), jnp.float32),
            pltpu.VMEM((_N, _B * _F1), jnp.float32),
            pltpu.VMEM((_N, _B * _F2), jnp.float32),
            pltpu.VMEM((_B * _F2, _C * _F2), jnp.float32),
        ],
        compiler_params=pltpu.CompilerParams(
            dimension_semantics=("arbitrary", "arbitrary"),
            vmem_limit_bytes=100 * 1024 * 1024,
        ),
    )(L, L, X0, M1, M2, b1t, b2t, Wfcf, msk, sb, sc, bfcr)
    return out


# speedup vs baseline: 2.8495x; 1.1707x over previous
import jax
import jax.numpy as jnp
from jax.experimental import pallas as pl
from jax.experimental.pallas import tpu as pltpu

_N = 4096
_B = 4
_K = 5
_F1 = 20
_F2 = 30
_C = 10
_BR = 256
_NI = _N // _BR

_HP = jax.lax.Precision.DEFAULT


def _mega_kernel(lhbm_ref, x0_ref, m1_ref, m2_ref, b1_ref, b2_ref,
                 wfc_ref, msk_ref, sb_ref, sc_ref, bfc_ref, out_ref,
                 lb, tbuf, hb, sa, sb_s, out2, uacc):
    p = pl.program_id(0)
    i = pl.program_id(1)
    r0 = i * _BR

    def lmul(full_f32):
        lrow = lb[pl.ds(r0, _BR), :]
        return jax.lax.dot_general(
            lrow, full_f32.astype(jnp.bfloat16),
            dimension_numbers=(((1,), (0,)), ((), ())),
            preferred_element_type=jnp.float32, precision=_HP)

    m1 = m1_ref[...]
    m2 = m2_ref[...]

    @pl.when(p == 0)
    def _():
        lblk = lhbm_ref[...]
        lb[pl.ds(r0, _BR), :] = lblk.astype(jnp.bfloat16)
        t1 = jax.lax.dot(lblk, x0_ref[...], precision=_HP)
        tbuf[pl.ds(r0, _BR), 0:4] = t1
        x0b = x0_ref[pl.ds(r0, _BR), :]
        hb[pl.ds(r0, _BR), :] = (jax.lax.dot(x0b, m1[0:4], precision=_HP)
                                 + jax.lax.dot(t1, m1[4:8], precision=_HP))

    @pl.when(p == 1)
    def _():
        t2 = (2.0 * lmul(tbuf[:, 0:4]) - x0_ref[pl.ds(r0, _BR), :])
        tbuf[pl.ds(r0, _BR), 4:8] = t2
        hb[pl.ds(r0, _BR), :] += jax.lax.dot(t2, m1[8:12], precision=_HP)

    @pl.when(p == 2)
    def _():
        t3 = 2.0 * lmul(tbuf[:, 4:8]) - tbuf[pl.ds(r0, _BR), 0:4]
        tbuf[pl.ds(r0, _BR), 8:12] = t3
        hb[pl.ds(r0, _BR), :] += jax.lax.dot(t3, m1[12:16], precision=_HP)

    @pl.when(p == 3)
    def _():
        t4 = 2.0 * lmul(tbuf[:, 8:12]) - tbuf[pl.ds(r0, _BR), 4:8]
        acc = hb[pl.ds(r0, _BR), :] + jax.lax.dot(t4, m1[16:20],
                                                  precision=_HP)
        hb[pl.ds(r0, _BR), :] = jnp.maximum(acc + b1_ref[...], 0.0)

    @pl.when(p == 4)
    def _():
        s1 = lmul(hb[...])
        sa[pl.ds(r0, _BR), :] = s1
        hblk = hb[pl.ds(r0, _BR), :]
        out2[pl.ds(r0, _BR), :] = (
            jax.lax.dot(hblk, m2[0:80], precision=_HP)
            + jax.lax.dot(s1, m2[80:160], precision=_HP))

    @pl.when(p == 5)
    def _():
        s2 = 2.0 * lmul(sa[...]) - hb[pl.ds(r0, _BR), :]
        sb_s[pl.ds(r0, _BR), :] = s2
        out2[pl.ds(r0, _BR), :] += jax.lax.dot(s2, m2[160:240], precision=_HP)

    @pl.when(p == 6)
    def _():
        s3 = 2.0 * lmul(sb_s[...]) - sa[pl.ds(r0, _BR), :]
        sa[pl.ds(r0, _BR), :] = s3
        out2[pl.ds(r0, _BR), :] += jax.lax.dot(s3, m2[240:320], precision=_HP)

    @pl.when(p == 7)
    def _():
        s4 = 2.0 * lmul(sa[...]) - sb_s[pl.ds(r0, _BR), :]
        acc = out2[pl.ds(r0, _BR), :] + jax.lax.dot(s4, m2[320:400],
                                                    precision=_HP)
        h2 = jnp.maximum(acc + b2_ref[...], 0.0)
        upart = jax.lax.dot_general(h2, wfc_ref[...],
                                    dimension_numbers=(((0,), (0,)), ((), ())),
                                    precision=_HP)

        @pl.when(i == 0)
        def _():
            uacc[...] = upart

        @pl.when(i > 0)
        def _():
            uacc[...] += upart

        @pl.when(i == _NI - 1)
        def _():
            um = uacc[...] * msk_ref[...]
            logits = jax.lax.dot(
                sb_ref[...], jax.lax.dot(um, sc_ref[...], precision=_HP),
                precision=_HP) + bfc_ref[...]
            m = jnp.max(logits, axis=1, keepdims=True)
            z = logits - m
            lse = jnp.log(jnp.sum(jnp.exp(z), axis=1, keepdims=True))
            out_ref[...] = z - lse


def kernel(x, L, W1, b1, W2, b2, Wfc, bfc):
    B, N = _B, _N
    X0 = x[:, :, 0].T

    eyeB = jnp.eye(B, dtype=jnp.float32)
    M1 = (W1[:, 0, :][:, None, None, :] * eyeB[None, :, :, None]
          ).reshape(_K * B, B * _F1)
    M2 = (W2[:, None, :, None, :] * eyeB[None, :, None, :, None]
          ).reshape(_K * B * _F1, B * _F2)
    b1t = jnp.tile(b1, (B,))[None, :]
    b2t = jnp.tile(b2, (B,))[None, :]

    Wfcf = Wfc.reshape(_C, N, _F2).transpose(1, 0, 2).reshape(N, _C * _F2)

    r = jnp.arange(B * _F2)[:, None]
    q = jnp.arange(_C * _F2)[None, :]
    msk = ((r % _F2) == (q % _F2)).astype(jnp.float32)
    sb = (jnp.arange(B)[:, None] == (jnp.arange(B * _F2)[None, :] // _F2)
          ).astype(jnp.float32)
    sc = ((jnp.arange(_C * _F2)[:, None] // _F2) == jnp.arange(_C)[None, :]
          ).astype(jnp.float32)
    bfcr = bfc[None, :]

    grid = (2 * _K - 2, _NI)
    out = pl.pallas_call(
        _mega_kernel,
        grid=grid,
        in_specs=[
            pl.BlockSpec((_BR, _N),
                         lambda p, i: (jnp.where(p == 0, i, 0), 0)),
            pl.BlockSpec((_N, _B), lambda p, i: (0, 0)),
            pl.BlockSpec((_K * _B, _B * _F1), lambda p, i: (0, 0)),
            pl.BlockSpec((_K * _B * _F1, _B * _F2), lambda p, i: (0, 0)),
            pl.BlockSpec((1, _B * _F1), lambda p, i: (0, 0)),
            pl.BlockSpec((1, _B * _F2), lambda p, i: (0, 0)),
            pl.BlockSpec((_BR, _C * _F2),
                         lambda p, i: (jnp.where(p == 7, i, 0), 0)),
            pl.BlockSpec((_B * _F2, _C * _F2), lambda p, i: (0, 0)),
            pl.BlockSpec((_B, _B * _F2), lambda p, i: (0, 0)),
            pl.BlockSpec((_C * _F2, _C), lambda p, i: (0, 0)),
            pl.BlockSpec((1, _C), lambda p, i: (0, 0)),
        ],
        out_specs=pl.BlockSpec((_B, _C), lambda p, i: (0, 0)),
        out_shape=jax.ShapeDtypeStruct((B, _C), jnp.float32),
        scratch_shapes=[
            pltpu.VMEM((_N, _N), jnp.bfloat16),
            pltpu.VMEM((_N, 16), jnp.float32),
            pltpu.VMEM((_N, _B * _F1), jnp.float32),
            pltpu.VMEM((_N, _B * _F1), jnp.float32),
            pltpu.VMEM((_N, _B * _F1), jnp.float32),
            pltpu.VMEM((_N, _B * _F2), jnp.float32),
            pltpu.VMEM((_B * _F2, _C * _F2), jnp.float32),
        ],
        compiler_params=pltpu.CompilerParams(
            dimension_semantics=("arbitrary", "arbitrary"),
            vmem_limit_bytes=100 * 1024 * 1024,
        ),
    )(L, X0, M1, M2, b1t, b2t, Wfcf, msk, sb, sc, bfcr)
    return out


# speedup vs baseline: 2.8624x; 1.0045x over previous
import jax
import jax.numpy as jnp
from jax.experimental import pallas as pl
from jax.experimental.pallas import tpu as pltpu

_N = 4096
_B = 4
_K = 5
_F1 = 20
_F2 = 30
_C = 10
_BR = 256
_NI = _N // _BR

_HP = jax.lax.Precision.DEFAULT
_BF = jnp.bfloat16


def _mega_kernel(lhbm_ref, x0_ref, m1_ref, m2_ref, b1_ref, b2_ref,
                 wfc_ref, msk_ref, sb_ref, sc_ref, bfc_ref, out_ref,
                 lb, tbuf, tb16, hb, hb16, sa, sa16, sb_s, sbs16,
                 out2, uacc):
    p = pl.program_id(0)
    i = pl.program_id(1)
    r0 = i * _BR

    def lmul(full_bf16):
        lrow = lb[pl.ds(r0, _BR), :]
        return jax.lax.dot_general(
            lrow, full_bf16,
            dimension_numbers=(((1,), (0,)), ((), ())),
            preferred_element_type=jnp.float32, precision=_HP)

    m1 = m1_ref[...]
    m2 = m2_ref[...]

    @pl.when(p == 0)
    def _():
        lblk = lhbm_ref[...]
        lb[pl.ds(r0, _BR), :] = lblk.astype(_BF)
        t1 = jax.lax.dot(lblk, x0_ref[...], precision=_HP)
        tbuf[pl.ds(r0, _BR), 0:4] = t1
        tb16[pl.ds(r0, _BR), 0:4] = t1.astype(_BF)
        x0b = x0_ref[pl.ds(r0, _BR), :]
        hb[pl.ds(r0, _BR), :] = (jax.lax.dot(x0b, m1[0:4], precision=_HP)
                                 + jax.lax.dot(t1, m1[4:8], precision=_HP))

    @pl.when(p == 1)
    def _():
        t2 = (2.0 * lmul(tb16[:, 0:4]) - x0_ref[pl.ds(r0, _BR), :])
        tbuf[pl.ds(r0, _BR), 4:8] = t2
        tb16[pl.ds(r0, _BR), 4:8] = t2.astype(_BF)
        hb[pl.ds(r0, _BR), :] += jax.lax.dot(t2, m1[8:12], precision=_HP)

    @pl.when(p == 2)
    def _():
        t3 = 2.0 * lmul(tb16[:, 4:8]) - tbuf[pl.ds(r0, _BR), 0:4]
        tbuf[pl.ds(r0, _BR), 8:12] = t3
        tb16[pl.ds(r0, _BR), 8:12] = t3.astype(_BF)
        hb[pl.ds(r0, _BR), :] += jax.lax.dot(t3, m1[12:16], precision=_HP)

    @pl.when(p == 3)
    def _():
        t4 = 2.0 * lmul(tb16[:, 8:12]) - tbuf[pl.ds(r0, _BR), 4:8]
        acc = hb[pl.ds(r0, _BR), :] + jax.lax.dot(t4, m1[16:20],
                                                  precision=_HP)
        h = jnp.maximum(acc + b1_ref[...], 0.0)
        hb[pl.ds(r0, _BR), :] = h
        hb16[pl.ds(r0, _BR), :] = h.astype(_BF)

    @pl.when(p == 4)
    def _():
        s1 = lmul(hb16[...])
        sa[pl.ds(r0, _BR), :] = s1
        sa16[pl.ds(r0, _BR), :] = s1.astype(_BF)
        hblk = hb[pl.ds(r0, _BR), :]
        out2[pl.ds(r0, _BR), :] = (
            jax.lax.dot(hblk, m2[0:80], precision=_HP)
            + jax.lax.dot(s1, m2[80:160], precision=_HP))

    @pl.when(p == 5)
    def _():
        s2 = 2.0 * lmul(sa16[...]) - hb[pl.ds(r0, _BR), :]
        sb_s[pl.ds(r0, _BR), :] = s2
        sbs16[pl.ds(r0, _BR), :] = s2.astype(_BF)
        out2[pl.ds(r0, _BR), :] += jax.lax.dot(s2, m2[160:240], precision=_HP)

    @pl.when(p == 6)
    def _():
        s3 = 2.0 * lmul(sbs16[...]) - sa[pl.ds(r0, _BR), :]
        sa[pl.ds(r0, _BR), :] = s3
        sa16[pl.ds(r0, _BR), :] = s3.astype(_BF)
        out2[pl.ds(r0, _BR), :] += jax.lax.dot(s3, m2[240:320], precision=_HP)

    @pl.when(p == 7)
    def _():
        s4 = 2.0 * lmul(sa16[...]) - sb_s[pl.ds(r0, _BR), :]
        acc = out2[pl.ds(r0, _BR), :] + jax.lax.dot(s4, m2[320:400],
                                                    precision=_HP)
        h2 = jnp.maximum(acc + b2_ref[...], 0.0)
        upart = jax.lax.dot_general(h2, wfc_ref[...],
                                    dimension_numbers=(((0,), (0,)), ((), ())),
                                    precision=_HP)

        @pl.when(i == 0)
        def _():
            uacc[...] = upart

        @pl.when(i > 0)
        def _():
            uacc[...] += upart

        @pl.when(i == _NI - 1)
        def _():
            um = uacc[...] * msk_ref[...]
            logits = jax.lax.dot(
                sb_ref[...], jax.lax.dot(um, sc_ref[...], precision=_HP),
                precision=_HP) + bfc_ref[...]
            m = jnp.max(logits, axis=1, keepdims=True)
            z = logits - m
            lse = jnp.log(jnp.sum(jnp.exp(z), axis=1, keepdims=True))
            out_ref[...] = z - lse


def kernel(x, L, W1, b1, W2, b2, Wfc, bfc):
    B, N = _B, _N
    X0 = x[:, :, 0].T

    eyeB = jnp.eye(B, dtype=jnp.float32)
    M1 = (W1[:, 0, :][:, None, None, :] * eyeB[None, :, :, None]
          ).reshape(_K * B, B * _F1)
    M2 = (W2[:, None, :, None, :] * eyeB[None, :, None, :, None]
          ).reshape(_K * B * _F1, B * _F2)
    b1t = jnp.tile(b1, (B,))[None, :]
    b2t = jnp.tile(b2, (B,))[None, :]

    Wfcf = Wfc.reshape(_C, N, _F2).transpose(1, 0, 2).reshape(N, _C * _F2)

    r = jnp.arange(B * _F2)[:, None]
    q = jnp.arange(_C * _F2)[None, :]
    msk = ((r % _F2) == (q % _F2)).astype(jnp.float32)
    sb = (jnp.arange(B)[:, None] == (jnp.arange(B * _F2)[None, :] // _F2)
          ).astype(jnp.float32)
    sc = ((jnp.arange(_C * _F2)[:, None] // _F2) == jnp.arange(_C)[None, :]
          ).astype(jnp.float32)
    bfcr = bfc[None, :]

    grid = (2 * _K - 2, _NI)
    out = pl.pallas_call(
        _mega_kernel,
        grid=grid,
        in_specs=[
            pl.BlockSpec((_BR, _N),
                         lambda p, i: (jnp.where(p == 0, i, 0), 0)),
            pl.BlockSpec((_N, _B), lambda p, i: (0, 0)),
            pl.BlockSpec((_K * _B, _B * _F1), lambda p, i: (0, 0)),
            pl.BlockSpec((_K * _B * _F1, _B * _F2), lambda p, i: (0, 0)),
            pl.BlockSpec((1, _B * _F1), lambda p, i: (0, 0)),
            pl.BlockSpec((1, _B * _F2), lambda p, i: (0, 0)),
            pl.BlockSpec((_BR, _C * _F2),
                         lambda p, i: (jnp.where(p == 7, i, 0), 0)),
            pl.BlockSpec((_B * _F2, _C * _F2), lambda p, i: (0, 0)),
            pl.BlockSpec((_B, _B * _F2), lambda p, i: (0, 0)),
            pl.BlockSpec((_C * _F2, _C), lambda p, i: (0, 0)),
            pl.BlockSpec((1, _C), lambda p, i: (0, 0)),
        ],
        out_specs=pl.BlockSpec((_B, _C), lambda p, i: (0, 0)),
        out_shape=jax.ShapeDtypeStruct((B, _C), jnp.float32),
        scratch_shapes=[
            pltpu.VMEM((_N, _N), _BF),
            pltpu.VMEM((_N, 16), jnp.float32),
            pltpu.VMEM((_N, 16), _BF),
            pltpu.VMEM((_N, _B * _F1), jnp.float32),
            pltpu.VMEM((_N, _B * _F1), _BF),
            pltpu.VMEM((_N, _B * _F1), jnp.float32),
            pltpu.VMEM((_N, _B * _F1), _BF),
            pltpu.VMEM((_N, _B * _F1), jnp.float32),
            pltpu.VMEM((_N, _B * _F1), _BF),
            pltpu.VMEM((_N, _B * _F2), jnp.float32),
            pltpu.VMEM((_B * _F2, _C * _F2), jnp.float32),
        ],
        compiler_params=pltpu.CompilerParams(
            dimension_semantics=("arbitrary", "arbitrary"),
            vmem_limit_bytes=100 * 1024 * 1024,
        ),
    )(L, X0, M1, M2, b1t, b2t, Wfcf, msk, sb, sc, bfcr)
    return out


# speedup vs baseline: 3.2943x; 1.1509x over previous
import jax
import jax.numpy as jnp
from jax.experimental import pallas as pl
from jax.experimental.pallas import tpu as pltpu

_N = 4096
_B = 4
_K = 5
_F1 = 20
_F2 = 30
_C = 10
_BR = 512
_NI = _N // _BR

_HP = jax.lax.Precision.DEFAULT
_BF = jnp.bfloat16


def _mega_kernel(lhbm_ref, x0_ref, m1_ref, m2_ref, b1_ref, b2_ref,
                 wfc_ref, msk_ref, sb_ref, sc_ref, bfc_ref, out_ref,
                 lb, tbuf, hb, sa, sb_s, out2, uacc):
    p = pl.program_id(0)
    i = pl.program_id(1)
    r0 = i * _BR

    def lmul(full_f32):
        lrow = lb[pl.ds(r0, _BR), :]
        return jax.lax.dot_general(
            lrow, full_f32.astype(_BF),
            dimension_numbers=(((1,), (0,)), ((), ())),
            preferred_element_type=jnp.float32, precision=_HP)

    m1 = m1_ref[...]
    m2 = m2_ref[...]

    @pl.when(p == 0)
    def _():
        lblk = lhbm_ref[...]
        lb[pl.ds(r0, _BR), :] = lblk.astype(_BF)
        t1 = jax.lax.dot(lblk, x0_ref[...], precision=_HP)
        tbuf[pl.ds(r0, _BR), 0:4] = t1
        x0b = x0_ref[pl.ds(r0, _BR), :]
        hb[pl.ds(r0, _BR), :] = (jax.lax.dot(x0b, m1[0:4], precision=_HP)
                                 + jax.lax.dot(t1, m1[4:8], precision=_HP))

    @pl.when(p == 1)
    def _():
        t2 = (2.0 * lmul(tbuf[:, 0:4]) - x0_ref[pl.ds(r0, _BR), :])
        tbuf[pl.ds(r0, _BR), 4:8] = t2
        hb[pl.ds(r0, _BR), :] += jax.lax.dot(t2, m1[8:12], precision=_HP)

    @pl.when(p == 2)
    def _():
        t3 = 2.0 * lmul(tbuf[:, 4:8]) - tbuf[pl.ds(r0, _BR), 0:4]
        tbuf[pl.ds(r0, _BR), 8:12] = t3
        hb[pl.ds(r0, _BR), :] += jax.lax.dot(t3, m1[12:16], precision=_HP)

    @pl.when(p == 3)
    def _():
        t4 = 2.0 * lmul(tbuf[:, 8:12]) - tbuf[pl.ds(r0, _BR), 4:8]
        acc = hb[pl.ds(r0, _BR), :] + jax.lax.dot(t4, m1[16:20],
                                                  precision=_HP)
        hb[pl.ds(r0, _BR), :] = jnp.maximum(acc + b1_ref[...], 0.0)

    @pl.when(p == 4)
    def _():
        s1 = lmul(hb[...])
        sa[pl.ds(r0, _BR), :] = s1
        hblk = hb[pl.ds(r0, _BR), :]
        out2[pl.ds(r0, _BR), :] = (
            jax.lax.dot(hblk, m2[0:80], precision=_HP)
            + jax.lax.dot(s1, m2[80:160], precision=_HP))

    @pl.when(p == 5)
    def _():
        s2 = 2.0 * lmul(sa[...]) - hb[pl.ds(r0, _BR), :]
        sb_s[pl.ds(r0, _BR), :] = s2
        out2[pl.ds(r0, _BR), :] += jax.lax.dot(s2, m2[160:240], precision=_HP)

    @pl.when(p == 6)
    def _():
        s3 = 2.0 * lmul(sb_s[...]) - sa[pl.ds(r0, _BR), :]
        sa[pl.ds(r0, _BR), :] = s3
        out2[pl.ds(r0, _BR), :] += jax.lax.dot(s3, m2[240:320], precision=_HP)

    @pl.when(p == 7)
    def _():
        s4 = 2.0 * lmul(sa[...]) - sb_s[pl.ds(r0, _BR), :]
        acc = out2[pl.ds(r0, _BR), :] + jax.lax.dot(s4, m2[320:400],
                                                    precision=_HP)
        h2 = jnp.maximum(acc + b2_ref[...], 0.0)
        upart = jax.lax.dot_general(h2, wfc_ref[...],
                                    dimension_numbers=(((0,), (0,)), ((), ())),
                                    precision=_HP)

        @pl.when(i == 0)
        def _():
            uacc[...] = upart

        @pl.when(i > 0)
        def _():
            uacc[...] += upart

        @pl.when(i == _NI - 1)
        def _():
            um = uacc[...] * msk_ref[...]
            logits = jax.lax.dot(
                sb_ref[...], jax.lax.dot(um, sc_ref[...], precision=_HP),
                precision=_HP) + bfc_ref[...]
            m = jnp.max(logits, axis=1, keepdims=True)
            z = logits - m
            lse = jnp.log(jnp.sum(jnp.exp(z), axis=1, keepdims=True))
            out_ref[...] = z - lse


def kernel(x, L, W1, b1, W2, b2, Wfc, bfc):
    B, N = _B, _N
    X0 = x[:, :, 0].T

    eyeB = jnp.eye(B, dtype=jnp.float32)
    M1 = (W1[:, 0, :][:, None, None, :] * eyeB[None, :, :, None]
          ).reshape(_K * B, B * _F1)
    M2 = (W2[:, None, :, None, :] * eyeB[None, :, None, :, None]
          ).reshape(_K * B * _F1, B * _F2)
    b1t = jnp.tile(b1, (B,))[None, :]
    b2t = jnp.tile(b2, (B,))[None, :]

    Wfcf = Wfc.reshape(_C, N, _F2).transpose(1, 0, 2).reshape(N, _C * _F2)

    r = jnp.arange(B * _F2)[:, None]
    q = jnp.arange(_C * _F2)[None, :]
    msk = ((r % _F2) == (q % _F2)).astype(jnp.float32)
    sb = (jnp.arange(B)[:, None] == (jnp.arange(B * _F2)[None, :] // _F2)
          ).astype(jnp.float32)
    sc = ((jnp.arange(_C * _F2)[:, None] // _F2) == jnp.arange(_C)[None, :]
          ).astype(jnp.float32)
    bfcr = bfc[None, :]

    grid = (2 * _K - 2, _NI)
    out = pl.pallas_call(
        _mega_kernel,
        grid=grid,
        in_specs=[
            pl.BlockSpec((_BR, _N),
                         lambda p, i: (jnp.where(p == 0, i, _NI - 1), 0)),
            pl.BlockSpec((_N, _B), lambda p, i: (0, 0)),
            pl.BlockSpec((_K * _B, _B * _F1), lambda p, i: (0, 0)),
            pl.BlockSpec((_K * _B * _F1, _B * _F2), lambda p, i: (0, 0)),
            pl.BlockSpec((1, _B * _F1), lambda p, i: (0, 0)),
            pl.BlockSpec((1, _B * _F2), lambda p, i: (0, 0)),
            pl.BlockSpec((_BR, _C * _F2),
                         lambda p, i: (jnp.where(p == 7, i, 0), 0)),
            pl.BlockSpec((_B * _F2, _C * _F2), lambda p, i: (0, 0)),
            pl.BlockSpec((_B, _B * _F2), lambda p, i: (0, 0)),
            pl.BlockSpec((_C * _F2, _C), lambda p, i: (0, 0)),
            pl.BlockSpec((1, _C), lambda p, i: (0, 0)),
        ],
        out_specs=pl.BlockSpec((_B, _C), lambda p, i: (0, 0)),
        out_shape=jax.ShapeDtypeStruct((B, _C), jnp.float32),
        scratch_shapes=[
            pltpu.VMEM((_N, _N), _BF),
            pltpu.VMEM((_N, 16), jnp.float32),
            pltpu.VMEM((_N, _B * _F1), jnp.float32),
            pltpu.VMEM((_N, _B * _F1), jnp.float32),
            pltpu.VMEM((_N, _B * _F1), jnp.float32),
            pltpu.VMEM((_N, _B * _F2), jnp.float32),
            pltpu.VMEM((_B * _F2, _C * _F2), jnp.float32),
        ],
        compiler_params=pltpu.CompilerParams(
            dimension_semantics=("arbitrary", "arbitrary"),
            vmem_limit_bytes=100 * 1024 * 1024,
        ),
    )(L, X0, M1, M2, b1t, b2t, Wfcf, msk, sb, sc, bfcr)
    return out
